# baseline (speedup 1.0000x reference)
import jax
import jax.numpy as jnp
from jax.experimental import pallas as pl
from jax.experimental.pallas import tpu as pltpu

_D = 1024
_H = 16
_DH = 64
_S = 2048
_K = 16
_QB = 512
_NEG = -1e30


def _qkv_proj_kernel(q_ref, k_ref, v_ref, wq_ref, wk_ref, wv_ref,
                     qt_ref, kt_ref, vt_ref):
    qt_ref[...] = jax.lax.dot_general(
        wq_ref[...], q_ref[...], (((0,), (1,)), ((), ())),
        preferred_element_type=jnp.float32)
    kt_ref[...] = jax.lax.dot_general(
        wk_ref[...], k_ref[...], (((0,), (1,)), ((), ())),
        preferred_element_type=jnp.float32)
    vt_ref[...] = jax.lax.dot_general(
        wv_ref[...].astype(jnp.bfloat16), v_ref[...].astype(jnp.bfloat16),
        (((0,), (1,)), ((), ())),
        preferred_element_type=jnp.float32).astype(jnp.bfloat16)


def _merge22(a, b):
    c1 = jnp.maximum(a[0], b[0])
    l1 = jnp.minimum(a[0], b[0])
    h2 = jnp.maximum(a[1], b[1])
    c4 = jnp.minimum(a[1], b[1])
    c2 = jnp.maximum(l1, h2)
    c3 = jnp.minimum(l1, h2)
    return (c1, c2, c3, c4)


def _merge44_top4(a, b):
    e1 = jnp.maximum(a[0], b[3])
    e2 = jnp.maximum(a[1], b[2])
    e3 = jnp.maximum(a[2], b[1])
    e4 = jnp.maximum(a[3], b[0])
    f1 = jnp.maximum(e1, e3)
    f3 = jnp.minimum(e1, e3)
    f2 = jnp.maximum(e2, e4)
    f4 = jnp.minimum(e2, e4)
    g1 = jnp.maximum(f1, f2)
    g2 = jnp.minimum(f1, f2)
    g3 = jnp.maximum(f3, f4)
    g4 = jnp.minimum(f3, f4)
    return (g1, g2, g3, g4)


def _topk_thresh(s):
    neg = jnp.float32(_NEG)
    chunks = [s[:, c * 128:(c + 1) * 128] for c in range(16)]
    s2 = [(jnp.maximum(chunks[2 * i], chunks[2 * i + 1]),
           jnp.minimum(chunks[2 * i], chunks[2 * i + 1])) for i in range(8)]
    s4 = [_merge22(s2[2 * i], s2[2 * i + 1]) for i in range(4)]
    t4 = [_merge44_top4(s4[2 * i], s4[2 * i + 1]) for i in range(2)]
    c1, c2, c3, c4 = _merge44_top4(t4[0], t4[1])

    a1, a2, a3, a4 = c1, c2, c3, c4
    rowmax = None
    thresh = None
    for i in range(_K):
        m = jnp.max(a1, axis=-1, keepdims=True)
        if i == 0:
            rowmax = m
        thresh = m
        if i < _K - 1:
            drop = a1 >= m
            a1 = jnp.where(drop, a2, a1)
            a2 = jnp.where(drop, a3, a2)
            a3 = jnp.where(drop, a4, a3)
            a4 = jnp.where(drop, neg, a4)

    acc = jnp.where(c1 >= thresh, jnp.exp(c1 - rowmax), 0.0)
    acc = acc + jnp.where(c2 >= thresh, jnp.exp(c2 - rowmax), 0.0)
    acc = acc + jnp.where(c3 >= thresh, jnp.exp(c3 - rowmax), 0.0)
    acc = acc + jnp.where(c4 >= thresh, jnp.exp(c4 - rowmax), 0.0)
    denom = jnp.sum(acc, axis=-1, keepdims=True)
    return rowmax, thresh, jnp.log(denom)


def _attn_kernel(qt_ref, kt_ref, vh_ref, attn_ref, ctx_ref):
    s = jax.lax.dot_general(
        qt_ref[...], kt_ref[...], (((0,), (0,)), ((), ())),
        preferred_element_type=jnp.float32)

    rowmax, thresh, logz = _topk_thresh(s)

    attn = jnp.where(s >= thresh, jnp.exp(s - (rowmax + logz)), 0.0)
    attn_ref[0, 0] = attn

    ctx_ref[...] = jax.lax.dot_general(
        vh_ref[...], attn.astype(jnp.bfloat16), (((1,), (1,)), ((), ())),
        preferred_element_type=jnp.float32).astype(jnp.bfloat16)


def _out_proj_kernel(ctx_ref, wo_ref, out_ref):
    out_ref[...] = jax.lax.dot_general(
        ctx_ref[...], wo_ref[...], (((0,), (0,)), ((), ())),
        preferred_element_type=jnp.float32)


def kernel(q, k, v, Wq, Wk, Wv, Wo):
    B, S, D = q.shape
    q2 = q.reshape(S, D)
    k2 = k.reshape(S, D)
    v2 = v.reshape(S, D)
    wqs = Wq * 0.125
    wor = Wo.astype(jnp.bfloat16)

    qt, kt, vt = pl.pallas_call(
        _qkv_proj_kernel,
        grid=(4,),
        in_specs=[
            pl.BlockSpec((S, D), lambda c: (0, 0)),
            pl.BlockSpec((S, D), lambda c: (0, 0)),
            pl.BlockSpec((S, D), lambda c: (0, 0)),
            pl.BlockSpec((D, D // 4), lambda c: (0, c)),
            pl.BlockSpec((D, D // 4), lambda c: (0, c)),
            pl.BlockSpec((D, D // 4), lambda c: (0, c)),
        ],
        out_specs=[
            pl.BlockSpec((D // 4, S), lambda c: (c, 0)),
            pl.BlockSpec((D // 4, S), lambda c: (c, 0)),
            pl.BlockSpec((D // 4, S), lambda c: (c, 0)),
        ],
        out_shape=[
            jax.ShapeDtypeStruct((D, S), jnp.float32),
            jax.ShapeDtypeStruct((D, S), jnp.float32),
            jax.ShapeDtypeStruct((D, S), jnp.bfloat16),
        ],
    )(q2, k2, v2, wqs, Wk, Wv)

    nqb = S // _QB
    attn, ctx = pl.pallas_call(
        _attn_kernel,
        grid=(nqb, _H),
        in_specs=[
            pl.BlockSpec((_DH, _QB), lambda qb, h: (h, qb)),
            pl.BlockSpec((_DH, S), lambda qb, h: (h, 0)),
            pl.BlockSpec((_DH, S), lambda qb, h: (h, 0)),
        ],
        out_specs=[
            pl.BlockSpec((1, 1, _QB, S), lambda qb, h: (0, h, qb, 0)),
            pl.BlockSpec((_DH, _QB), lambda qb, h: (h, qb)),
        ],
        out_shape=[
            jax.ShapeDtypeStruct((1, _H, S, S), jnp.float32),
            jax.ShapeDtypeStruct((_H * _DH, S), jnp.bfloat16),
        ],
        compiler_params=pltpu.CompilerParams(
            dimension_semantics=("arbitrary", "arbitrary")),
    )(qt, kt, vt)

    out = pl.pallas_call(
        _out_proj_kernel,
        grid=(nqb,),
        in_specs=[
            pl.BlockSpec((_H * _DH, _QB), lambda qb: (0, qb)),
            pl.BlockSpec((_H * _DH, D), lambda qb: (0, 0)),
        ],
        out_specs=pl.BlockSpec((_QB, D), lambda qb: (qb, 0)),
        out_shape=jax.ShapeDtypeStruct((S, D), jnp.float32),
    )(ctx, wor)

    return out.reshape(B, S, D), attn
